# Initial kernel scaffold; baseline (speedup 1.0000x reference)
import jax
import jax.numpy as jnp
from jax.experimental import pallas as pl


def kernel(x, edge_index_rel0, edge_index_rel1, message_, W0, a_src0, a_dst0, W1, a_src1, a_dst1, b):
    raise NotImplementedError("write your pallas kernel here")


# speedup vs baseline: 14.7316x; 14.7316x over previous
import functools
import jax
import jax.numpy as jnp
from jax import lax
from jax.experimental import pallas as pl
from jax.experimental.pallas import tpu as pltpu
from jax.experimental.pallas import tpu_sc as plsc

N = 10000
D = 128
E = 160000

NC = 2
NS = 16
NW = NC * NS
EPT = 5120
E_PAD = NW * EPT
N_PAD = 10240
CHUNK = 128
NCHUNK = EPT // CHUNK
GRP = 16
GPC = CHUNK // GRP
NPT = N_PAD // NS
DEN_R = N_PAD // D

_f32 = jnp.float32
_i32 = jnp.int32


def _front_body(x_ref, w0_ref, w1_ref, a0_ref, a1_ref,
                z0_ref, z1_ref, s0_ref, s1_ref):
    xb = x_ref[...]
    z0 = jnp.dot(xb, w0_ref[...], preferred_element_type=_f32)
    z1 = jnp.dot(xb, w1_ref[...], preferred_element_type=_f32)
    z0_ref[...] = z0
    z1_ref[...] = z1
    s0_ref[...] = jnp.dot(z0, a0_ref[...], preferred_element_type=_f32)
    s1_ref[...] = jnp.dot(z1, a1_ref[...], preferred_element_type=_f32)


def _tc_front(x, W0, W1, A0, A1):
    blk = 2000
    grid = N // blk
    return pl.pallas_call(
        _front_body,
        grid=(grid,),
        in_specs=[
            pl.BlockSpec((blk, D), lambda i: (i, 0)),
            pl.BlockSpec((D, D), lambda i: (0, 0)),
            pl.BlockSpec((D, D), lambda i: (0, 0)),
            pl.BlockSpec((D, 2), lambda i: (0, 0)),
            pl.BlockSpec((D, 2), lambda i: (0, 0)),
        ],
        out_specs=[
            pl.BlockSpec((blk, D), lambda i: (i, 0)),
            pl.BlockSpec((blk, D), lambda i: (i, 0)),
            pl.BlockSpec((blk, 2), lambda i: (i, 0)),
            pl.BlockSpec((blk, 2), lambda i: (i, 0)),
        ],
        out_shape=[
            jax.ShapeDtypeStruct((N, D), _f32),
            jax.ShapeDtypeStruct((N, D), _f32),
            jax.ShapeDtypeStruct((N, 2), _f32),
            jax.ShapeDtypeStruct((N, 2), _f32),
        ],
    )(x, W0, W1, A0, A1)


def _lrelu(v):
    return jnp.where(v >= 0.0, v, 0.2 * v)


def _sc_body(z0_hbm, z1_hbm, ss0_hbm, sd0_hbm, ss1_hbm, sd1_hbm, src_hbm, dst_hbm,
             u_hbm, den_hbm,
             s_src_v, s_dst_v, src_v, dst_v, ex_c, rows_v, u_sh, den_sh):
    cid = lax.axis_index("c")
    sid = lax.axis_index("s")
    wid = cid * NS + sid
    row0 = sid * NPT
    iota = lax.iota(_i32, GRP)
    zeros16 = jnp.zeros((GRP,), _f32)

    def _zrow(i, _):
        for j in range(GPC):
            rows_v[i, pl.ds(j * GRP, GRP)] = zeros16
        return 0

    for rel in range(2):
        z_hbm = (z0_hbm, z1_hbm)[rel]
        ss_hbm = (ss0_hbm, ss1_hbm)[rel]
        sd_hbm = (sd0_hbm, sd1_hbm)[rel]

        lax.fori_loop(0, CHUNK, _zrow, 0)
        for g in range(GPC):
            ex_c[pl.ds(g * GRP, GRP)] = zeros16
        for k in range(NPT // CHUNK):
            pltpu.sync_copy(rows_v, u_sh.at[pl.ds(row0 + k * CHUNK, CHUNK)])
            pltpu.sync_copy(ex_c, den_sh.at[pl.ds(row0 + k * CHUNK, CHUNK)])

        pltpu.sync_copy(ss_hbm, s_src_v)
        pltpu.sync_copy(sd_hbm, s_dst_v)
        pltpu.sync_copy(src_hbm.at[rel, wid], src_v)
        pltpu.sync_copy(dst_hbm.at[rel, wid], dst_v)

        def _mx(i, c):
            a, bm = c
            return (jnp.maximum(a, s_src_v[pl.ds(i * GRP, GRP)]),
                    jnp.maximum(bm, s_dst_v[pl.ds(i * GRP, GRP)]))
        neg = jnp.full((GRP,), -3.0e38, _f32)
        mS, mD = lax.fori_loop(0, N // GRP, _mx, (neg, neg))

        def _allmax(v):
            for sh in (1, 2, 4, 8):
                ex_c[pl.ds(0, GRP)] = v
                v = jnp.maximum(
                    v, plsc.load_gather(ex_c, [jnp.bitwise_xor(iota, sh)]))
            return v
        mstar = _lrelu(_allmax(mS) + _allmax(mD))

        plsc.subcore_barrier()

        ebase = wid * EPT

        def _chunk(c, _):
            def _grp(g, _):
                sv = src_v[c, pl.ds(g * GRP, GRP)]
                dv = dst_v[c, pl.ds(g * GRP, GRP)]
                a = plsc.load_gather(s_src_v, [sv])
                bm = plsc.load_gather(s_dst_v, [dv])
                e = _lrelu(a + bm)
                ex = jnp.exp(e - mstar)
                valid = (ebase + c * CHUNK + g * GRP + iota) < E
                ex = jnp.where(valid, ex, 0.0)
                ex_c[pl.ds(g * GRP, GRP)] = ex
                return 0
            lax.fori_loop(0, GPC, _grp, 0)

            pltpu.sync_copy(ex_c, den_sh.at[dst_v.at[c]], add=True)

            pltpu.sync_copy(z_hbm.at[src_v.at[c]], rows_v)

            def _scale(i, _):
                w = plsc.load_gather(ex_c, [jnp.full((GRP,), i, _i32)])
                for j in range(GPC):
                    sl = pl.ds(j * GRP, GRP)
                    rows_v[i, sl] = rows_v[i, sl] * w
                return 0
            lax.fori_loop(0, CHUNK, _scale, 0)

            pltpu.sync_copy(rows_v, u_sh.at[dst_v.at[c]], add=True)
            return 0

        lax.fori_loop(0, NCHUNK, _chunk, 0)

        plsc.subcore_barrier()

        pltpu.sync_copy(u_sh.at[pl.ds(row0, NPT)],
                        u_hbm.at[rel, cid, pl.ds(row0, NPT)])
        pltpu.sync_copy(den_sh.at[pl.ds(row0, NPT)],
                        den_hbm.at[rel, cid, pl.ds(row0, NPT)])


def _sc_gat(z0, z1, ss0, sd0, ss1, sd1, src_r, dst_r):
    mesh = plsc.VectorSubcoreMesh(core_axis_name="c", subcore_axis_name="s",
                                  num_cores=NC, num_subcores=NS)
    f = pl.kernel(
        _sc_body,
        out_type=[
            jax.ShapeDtypeStruct((2, NC, N_PAD, D), _f32),
            jax.ShapeDtypeStruct((2, NC, N_PAD), _f32),
        ],
        mesh=mesh,
        compiler_params=pltpu.CompilerParams(needs_layout_passes=False),
        scratch_types=[
            pltpu.VMEM((N,), _f32),
            pltpu.VMEM((N,), _f32),
            pltpu.VMEM((NCHUNK, CHUNK), _i32),
            pltpu.VMEM((NCHUNK, CHUNK), _i32),
            pltpu.VMEM((CHUNK,), _f32),
            pltpu.VMEM((CHUNK, D), _f32),
            pltpu.VMEM_SHARED((N_PAD, D), _f32),
            pltpu.VMEM_SHARED((N_PAD,), _f32),
        ],
    )
    return f(z0, z1, ss0, sd0, ss1, sd1, src_r, dst_r)


def _tail_body(u_ref, den_ref, b_ref, o_ref):
    blk = o_ref.shape[0]
    den = jnp.sum(den_ref[...], axis=1)
    den = den.reshape(2, blk)
    den = jnp.where(den == 0.0, 1.0, den)
    u = u_ref[...]
    acc = (u[0, 0] + u[0, 1]) / den[0][:, None]
    acc = acc + (u[1, 0] + u[1, 1]) / den[1][:, None]
    o_ref[...] = acc + b_ref[...]


def _tc_tail(u_part, den_part, b):
    blk = 2048
    grid = N_PAD // blk
    return pl.pallas_call(
        _tail_body,
        grid=(grid,),
        in_specs=[
            pl.BlockSpec((2, NC, blk, D), lambda i: (0, 0, i, 0)),
            pl.BlockSpec((2, NC, blk // D, D), lambda i: (0, 0, i, 0)),
            pl.BlockSpec((1, D), lambda i: (0, 0)),
        ],
        out_specs=pl.BlockSpec((blk, D), lambda i: (i, 0)),
        out_shape=jax.ShapeDtypeStruct((N_PAD, D), _f32),
    )(u_part, den_part, b)


@jax.jit
def kernel(x, edge_index_rel0, edge_index_rel1, message_, W0, a_src0,
           a_dst0, W1, a_src1, a_dst1, b):
    A0 = jnp.stack([a_src0, a_dst0], axis=1)
    A1 = jnp.stack([a_src1, a_dst1], axis=1)
    z0, z1, s0p, s1p = _tc_front(x, W0, W1, A0, A1)
    ss0 = s0p[:, 0] + 0.0
    sd0 = s0p[:, 1] + 0.0
    ss1 = s1p[:, 0] + 0.0
    sd1 = s1p[:, 1] + 0.0

    pad = E_PAD - E
    ei = jnp.stack([edge_index_rel0, edge_index_rel1])
    ei = jnp.pad(ei, ((0, 0), (0, 0), (0, pad)))
    src_r = ei[:, 0].reshape(2, NW, NCHUNK, CHUNK)
    dst_r = ei[:, 1].reshape(2, NW, NCHUNK, CHUNK)

    u_part, den_part = _sc_gat(z0, z1, ss0, sd0, ss1, sd1, src_r, dst_r)
    den_part = den_part.reshape(2, NC, DEN_R, D)
    return _tc_tail(u_part, den_part, b.reshape(1, D))[:N]


# speedup vs baseline: 17.6485x; 1.1980x over previous
import functools
import jax
import jax.numpy as jnp
from jax import lax
from jax.experimental import pallas as pl
from jax.experimental.pallas import tpu as pltpu
from jax.experimental.pallas import tpu_sc as plsc

N = 10000
D = 128
E = 160000

NC = 2
NS = 16
NW = NC * NS
EPT = 5120
E_PAD = NW * EPT
N_PAD = 10240
CHUNK = 64
NQ = 4
QCH = EPT // (NQ * CHUNK)
NP = QCH // 2
GRP = 16
GPC = CHUNK // GRP
DG = 128 // GRP
NPT = N_PAD // NS
DEN_R = N_PAD // D

_f32 = jnp.float32
_i32 = jnp.int32


def _front_body(x_ref, w0_ref, w1_ref, a0_ref, a1_ref,
                z0_ref, z1_ref, s0_ref, s1_ref):
    xb = x_ref[...]
    z0 = jnp.dot(xb, w0_ref[...], preferred_element_type=_f32)
    z1 = jnp.dot(xb, w1_ref[...], preferred_element_type=_f32)
    z0_ref[...] = z0
    z1_ref[...] = z1
    s0_ref[...] = jnp.dot(z0, a0_ref[...], preferred_element_type=_f32)
    s1_ref[...] = jnp.dot(z1, a1_ref[...], preferred_element_type=_f32)


def _tc_front(x, W0, W1, A0, A1):
    blk = 2000
    grid = N // blk
    return pl.pallas_call(
        _front_body,
        grid=(grid,),
        in_specs=[
            pl.BlockSpec((blk, D), lambda i: (i, 0)),
            pl.BlockSpec((D, D), lambda i: (0, 0)),
            pl.BlockSpec((D, D), lambda i: (0, 0)),
            pl.BlockSpec((D, 2), lambda i: (0, 0)),
            pl.BlockSpec((D, 2), lambda i: (0, 0)),
        ],
        out_specs=[
            pl.BlockSpec((blk, D), lambda i: (i, 0)),
            pl.BlockSpec((blk, D), lambda i: (i, 0)),
            pl.BlockSpec((blk, 2), lambda i: (i, 0)),
            pl.BlockSpec((blk, 2), lambda i: (i, 0)),
        ],
        out_shape=[
            jax.ShapeDtypeStruct((N, D), _f32),
            jax.ShapeDtypeStruct((N, D), _f32),
            jax.ShapeDtypeStruct((N, 2), _f32),
            jax.ShapeDtypeStruct((N, 2), _f32),
        ],
    )(x, W0, W1, A0, A1)


def _lrelu(v):
    return jnp.where(v >= 0.0, v, 0.2 * v)


def _sc_body(z0_hbm, z1_hbm, ss0_hbm, sd0_hbm, ss1_hbm, sd1_hbm, pk_hbm,
             u_hbm, den_hbm,
             s_src_v, s_dst_v, pk_v, srcb_a, dstb_a, srcb_b, dstb_b,
             ex_a, ex_b, rows_a, rows_b, sem_ga, sem_gb, u_sh, den_sh):
    cid = lax.axis_index("c")
    sid = lax.axis_index("s")
    wid = cid * NS + sid
    row0 = sid * NPT
    iota = lax.iota(_i32, GRP)
    zeros16 = jnp.zeros((GRP,), _f32)

    def _zrow(i, _):
        for j in range(DG):
            rows_a[i, pl.ds(j * GRP, GRP)] = zeros16
        return 0

    for rel in range(2):
        z_hbm = (z0_hbm, z1_hbm)[rel]
        ss_hbm = (ss0_hbm, ss1_hbm)[rel]
        sd_hbm = (sd0_hbm, sd1_hbm)[rel]

        lax.fori_loop(0, CHUNK, _zrow, 0)
        for g in range(GPC):
            ex_a[pl.ds(g * GRP, GRP)] = zeros16
        for k in range(NPT // CHUNK):
            pltpu.sync_copy(rows_a, u_sh.at[pl.ds(row0 + k * CHUNK, CHUNK)])
            pltpu.sync_copy(ex_a, den_sh.at[pl.ds(row0 + k * CHUNK, CHUNK)])

        pltpu.sync_copy(ss_hbm, s_src_v)
        pltpu.sync_copy(sd_hbm, s_dst_v)

        def _mx(i, c):
            a, bm = c
            return (jnp.maximum(a, s_src_v[pl.ds(i * GRP, GRP)]),
                    jnp.maximum(bm, s_dst_v[pl.ds(i * GRP, GRP)]))
        neg = jnp.full((GRP,), -3.0e38, _f32)
        mS, mD = lax.fori_loop(0, N // GRP, _mx, (neg, neg))

        def _allmax(v):
            for sh in (1, 2, 4, 8):
                ex_a[pl.ds(0, GRP)] = v
                v = jnp.maximum(
                    v, plsc.load_gather(ex_a, [jnp.bitwise_xor(iota, sh)]))
            return v
        mstar = _lrelu(_allmax(mS) + _allmax(mD))

        plsc.subcore_barrier()

        for q in range(NQ):
            pltpu.sync_copy(pk_hbm.at[rel, wid, q], pk_v)
            qbase = wid * EPT + q * QCH * CHUNK

            def _score(c, exb, srcb, dstb):
                def _grp(g, _):
                    pv = pk_v[c, pl.ds(g * GRP, GRP)]
                    sv = lax.shift_right_logical(pv, 14)
                    dv = jnp.bitwise_and(pv, 16383)
                    srcb[pl.ds(g * GRP, GRP)] = sv
                    dstb[pl.ds(g * GRP, GRP)] = dv
                    a = plsc.load_gather(s_src_v, [sv])
                    bm = plsc.load_gather(s_dst_v, [dv])
                    e = _lrelu(a + bm)
                    ex = jnp.exp(e - mstar)
                    valid = (qbase + c * CHUNK + g * GRP + iota) < E
                    exb[pl.ds(g * GRP, GRP)] = jnp.where(valid, ex, 0.0)
                    return 0
                lax.fori_loop(0, GPC, _grp, 0)
                pltpu.sync_copy(exb, den_sh.at[dstb], add=True)

            def _scale(rows, exb):
                def _s(i, _):
                    w = plsc.load_gather(exb, [jnp.full((GRP,), i, _i32)])
                    for j in range(DG):
                        sl = pl.ds(j * GRP, GRP)
                        rows[i, sl] = rows[i, sl] * w
                    return 0
                lax.fori_loop(0, CHUNK, _s, 0)

            _score(0, ex_a, srcb_a, dstb_a)
            pltpu.async_copy(z_hbm.at[srcb_a], rows_a, sem_ga)
            _score(1, ex_b, srcb_b, dstb_b)
            pltpu.async_copy(z_hbm.at[srcb_b], rows_b, sem_gb)

            def _pair(k, _):
                a = 2 * k
                b = a + 1
                pltpu.make_async_copy(z_hbm.at[srcb_a], rows_a, sem_ga).wait()
                _scale(rows_a, ex_a)
                pltpu.async_copy(rows_a, u_sh.at[dstb_a], sem_ga, add=True)

                pltpu.make_async_copy(z_hbm.at[srcb_b], rows_b, sem_gb).wait()
                _scale(rows_b, ex_b)
                pltpu.async_copy(rows_b, u_sh.at[dstb_b], sem_gb, add=True)

                @pl.when(k < NP - 1)
                def _():
                    pltpu.make_async_copy(rows_a, u_sh.at[dstb_a],
                                          sem_ga).wait()
                    _score(a + 2, ex_a, srcb_a, dstb_a)
                    pltpu.async_copy(z_hbm.at[srcb_a], rows_a, sem_ga)
                    pltpu.make_async_copy(rows_b, u_sh.at[dstb_b],
                                          sem_gb).wait()
                    _score(b + 2, ex_b, srcb_b, dstb_b)
                    pltpu.async_copy(z_hbm.at[srcb_b], rows_b, sem_gb)
                return 0

            lax.fori_loop(0, NP, _pair, 0)

            pltpu.make_async_copy(rows_a, u_sh.at[dstb_a], sem_ga).wait()
            pltpu.make_async_copy(rows_b, u_sh.at[dstb_b], sem_gb).wait()

        plsc.subcore_barrier()

        pltpu.sync_copy(u_sh.at[pl.ds(row0, NPT)],
                        u_hbm.at[rel, cid, pl.ds(row0, NPT)])
        pltpu.sync_copy(den_sh.at[pl.ds(row0, NPT)],
                        den_hbm.at[rel, cid, pl.ds(row0, NPT)])


def _sc_gat(z0, z1, ss0, sd0, ss1, sd1, pk_r):
    mesh = plsc.VectorSubcoreMesh(core_axis_name="c", subcore_axis_name="s",
                                  num_cores=NC, num_subcores=NS)
    f = pl.kernel(
        _sc_body,
        out_type=[
            jax.ShapeDtypeStruct((2, NC, N_PAD, D), _f32),
            jax.ShapeDtypeStruct((2, NC, N_PAD), _f32),
        ],
        mesh=mesh,
        compiler_params=pltpu.CompilerParams(needs_layout_passes=False),
        scratch_types=[
            pltpu.VMEM((N,), _f32),
            pltpu.VMEM((N,), _f32),
            pltpu.VMEM((QCH, CHUNK), _i32),
            pltpu.VMEM((CHUNK,), _i32),
            pltpu.VMEM((CHUNK,), _i32),
            pltpu.VMEM((CHUNK,), _i32),
            pltpu.VMEM((CHUNK,), _i32),
            pltpu.VMEM((CHUNK,), _f32),
            pltpu.VMEM((CHUNK,), _f32),
            pltpu.VMEM((CHUNK, D), _f32),
            pltpu.VMEM((CHUNK, D), _f32),
            pltpu.SemaphoreType.DMA,
            pltpu.SemaphoreType.DMA,
            pltpu.VMEM_SHARED((N_PAD, D), _f32),
            pltpu.VMEM_SHARED((N_PAD,), _f32),
        ],
    )
    return f(z0, z1, ss0, sd0, ss1, sd1, pk_r)


def _tail_body(u_ref, den_ref, b_ref, o_ref):
    blk = o_ref.shape[0]
    den = jnp.sum(den_ref[...], axis=1)
    den = den.reshape(2, blk)
    den = jnp.where(den == 0.0, 1.0, den)
    u = u_ref[...]
    acc = (u[0, 0] + u[0, 1]) / den[0][:, None]
    acc = acc + (u[1, 0] + u[1, 1]) / den[1][:, None]
    o_ref[...] = acc + b_ref[...]


def _tc_tail(u_part, den_part, b):
    blk = 2048
    grid = N_PAD // blk
    return pl.pallas_call(
        _tail_body,
        grid=(grid,),
        in_specs=[
            pl.BlockSpec((2, NC, blk, D), lambda i: (0, 0, i, 0)),
            pl.BlockSpec((2, NC, blk // D, D), lambda i: (0, 0, i, 0)),
            pl.BlockSpec((1, D), lambda i: (0, 0)),
        ],
        out_specs=pl.BlockSpec((blk, D), lambda i: (i, 0)),
        out_shape=jax.ShapeDtypeStruct((N_PAD, D), _f32),
    )(u_part, den_part, b)


@jax.jit
def kernel(x, edge_index_rel0, edge_index_rel1, message_, W0, a_src0,
           a_dst0, W1, a_src1, a_dst1, b):
    A0 = jnp.stack([a_src0, a_dst0], axis=1)
    A1 = jnp.stack([a_src1, a_dst1], axis=1)
    z0, z1, s0p, s1p = _tc_front(x, W0, W1, A0, A1)
    ss0 = s0p[:, 0] + 0.0
    sd0 = s0p[:, 1] + 0.0
    ss1 = s1p[:, 0] + 0.0
    sd1 = s1p[:, 1] + 0.0

    pad = E_PAD - E
    ei = jnp.stack([edge_index_rel0, edge_index_rel1])
    ei = jnp.pad(ei, ((0, 0), (0, 0), (0, pad)))
    pk = (ei[:, 0] << 14) | ei[:, 1]
    pk_r = pk.reshape(2, NW, NQ, QCH, CHUNK)

    u_part, den_part = _sc_gat(z0, z1, ss0, sd0, ss1, sd1, pk_r)
    den_part = den_part.reshape(2, NC, DEN_R, D)
    return _tc_tail(u_part, den_part, b.reshape(1, D))[:N]


# speedup vs baseline: 18.6114x; 1.0546x over previous
import functools
import jax
import jax.numpy as jnp
from jax import lax
from jax.experimental import pallas as pl
from jax.experimental.pallas import tpu as pltpu
from jax.experimental.pallas import tpu_sc as plsc

N = 10000
D = 128
E = 160000

NC = 2
NS = 16
NW = NC * NS
EPT = 5120
E_PAD = NW * EPT
N_PAD = 10240
CHUNK = 32
NBUF = 4
NQ = 4
QCH = EPT // (NQ * CHUNK)
NP = QCH // NBUF
GRP = 16
GPC = CHUNK // GRP
DG = 128 // GRP
NPT = N_PAD // NS
DEN_R = N_PAD // D

_f32 = jnp.float32
_i32 = jnp.int32


def _front_body(x_ref, w0_ref, w1_ref, a0_ref, a1_ref,
                z0_ref, z1_ref, s0_ref, s1_ref):
    xb = x_ref[...]
    z0 = jnp.dot(xb, w0_ref[...], preferred_element_type=_f32)
    z1 = jnp.dot(xb, w1_ref[...], preferred_element_type=_f32)
    z0_ref[...] = z0
    z1_ref[...] = z1
    s0_ref[...] = jnp.dot(z0, a0_ref[...], preferred_element_type=_f32)
    s1_ref[...] = jnp.dot(z1, a1_ref[...], preferred_element_type=_f32)


def _tc_front(x, W0, W1, A0, A1):
    blk = 2000
    grid = N // blk
    return pl.pallas_call(
        _front_body,
        grid=(grid,),
        in_specs=[
            pl.BlockSpec((blk, D), lambda i: (i, 0)),
            pl.BlockSpec((D, D), lambda i: (0, 0)),
            pl.BlockSpec((D, D), lambda i: (0, 0)),
            pl.BlockSpec((D, 2), lambda i: (0, 0)),
            pl.BlockSpec((D, 2), lambda i: (0, 0)),
        ],
        out_specs=[
            pl.BlockSpec((blk, D), lambda i: (i, 0)),
            pl.BlockSpec((blk, D), lambda i: (i, 0)),
            pl.BlockSpec((blk, 2), lambda i: (i, 0)),
            pl.BlockSpec((blk, 2), lambda i: (i, 0)),
        ],
        out_shape=[
            jax.ShapeDtypeStruct((N, D), _f32),
            jax.ShapeDtypeStruct((N, D), _f32),
            jax.ShapeDtypeStruct((N, 2), _f32),
            jax.ShapeDtypeStruct((N, 2), _f32),
        ],
    )(x, W0, W1, A0, A1)


def _lrelu(v):
    return jnp.where(v >= 0.0, v, 0.2 * v)


def _sc_body(z0_hbm, z1_hbm, ss0_hbm, sd0_hbm, ss1_hbm, sd1_hbm, pk_hbm,
             u_hbm, den_hbm,
             s_src_v, s_dst_v, pk_v, srcb, dstb, exb, rows, sems,
             u_sh, den_sh):
    cid = lax.axis_index("c")
    sid = lax.axis_index("s")
    wid = cid * NS + sid
    row0 = sid * NPT
    iota = lax.iota(_i32, GRP)
    zeros16 = jnp.zeros((GRP,), _f32)
    rows0 = rows[0]

    def _zrow(i, _):
        for j in range(DG):
            rows0[i, pl.ds(j * GRP, GRP)] = zeros16
        return 0

    for rel in range(2):
        z_hbm = (z0_hbm, z1_hbm)[rel]
        ss_hbm = (ss0_hbm, ss1_hbm)[rel]
        sd_hbm = (sd0_hbm, sd1_hbm)[rel]

        lax.fori_loop(0, CHUNK, _zrow, 0)
        ex0 = exb[0]
        for g in range(GPC):
            ex0[pl.ds(g * GRP, GRP)] = zeros16
        for k in range(NPT // CHUNK):
            pltpu.sync_copy(rows0, u_sh.at[pl.ds(row0 + k * CHUNK, CHUNK)])
            pltpu.sync_copy(ex0, den_sh.at[pl.ds(row0 + k * CHUNK, CHUNK)])

        pltpu.sync_copy(ss_hbm, s_src_v)
        pltpu.sync_copy(sd_hbm, s_dst_v)

        def _mx(i, c):
            a, bm = c
            return (jnp.maximum(a, s_src_v[pl.ds(i * GRP, GRP)]),
                    jnp.maximum(bm, s_dst_v[pl.ds(i * GRP, GRP)]))
        neg = jnp.full((GRP,), -3.0e38, _f32)
        mS, mD = lax.fori_loop(0, N // GRP, _mx, (neg, neg))

        def _allmax(v):
            for sh in (1, 2, 4, 8):
                ex0[pl.ds(0, GRP)] = v
                v = jnp.maximum(
                    v, plsc.load_gather(ex0, [jnp.bitwise_xor(iota, sh)]))
            return v
        mstar = _lrelu(_allmax(mS) + _allmax(mD))

        plsc.subcore_barrier()

        for q in range(NQ):
            pltpu.sync_copy(pk_hbm.at[rel, wid, q], pk_v)
            qbase = wid * EPT + q * QCH * CHUNK

            def _score(c, i):
                def _grp(g, _):
                    pv = pk_v[c, pl.ds(g * GRP, GRP)]
                    sv = lax.shift_right_logical(pv, 14)
                    dv = jnp.bitwise_and(pv, 16383)
                    srcb[i][pl.ds(g * GRP, GRP)] = sv
                    dstb[i][pl.ds(g * GRP, GRP)] = dv
                    a = plsc.load_gather(s_src_v, [sv])
                    bm = plsc.load_gather(s_dst_v, [dv])
                    e = _lrelu(a + bm)
                    ex = jnp.exp(e - mstar)
                    valid = (qbase + c * CHUNK + g * GRP + iota) < E
                    exb[i][pl.ds(g * GRP, GRP)] = jnp.where(valid, ex, 0.0)
                    return 0
                lax.fori_loop(0, GPC, _grp, 0)
                pltpu.sync_copy(exb[i], den_sh.at[dstb[i]], add=True)

            def _scale(i):
                def _s(r, _):
                    w = plsc.load_gather(exb[i], [jnp.full((GRP,), r, _i32)])
                    for j in range(DG):
                        sl = pl.ds(j * GRP, GRP)
                        rows[i][r, sl] = rows[i][r, sl] * w
                    return 0
                lax.fori_loop(0, CHUNK, _s, 0)

            for i in range(NBUF):
                _score(i, i)
                pltpu.async_copy(z_hbm.at[srcb[i]], rows[i], sems[i])

            def _round(k, _):
                c0 = NBUF * k
                for i in range(NBUF):
                    pltpu.make_async_copy(z_hbm.at[srcb[i]], rows[i],
                                          sems[i]).wait()
                    _scale(i)
                    pltpu.async_copy(rows[i], u_sh.at[dstb[i]], sems[i],
                                     add=True)
                @pl.when(k < NP - 1)
                def _():
                    for i in range(NBUF):
                        pltpu.make_async_copy(rows[i], u_sh.at[dstb[i]],
                                              sems[i]).wait()
                        _score(c0 + NBUF + i, i)
                        pltpu.async_copy(z_hbm.at[srcb[i]], rows[i], sems[i])
                return 0

            lax.fori_loop(0, NP, _round, 0)

            for i in range(NBUF):
                pltpu.make_async_copy(rows[i], u_sh.at[dstb[i]],
                                      sems[i]).wait()

        plsc.subcore_barrier()

        pltpu.sync_copy(u_sh.at[pl.ds(row0, NPT)],
                        u_hbm.at[rel, cid, pl.ds(row0, NPT)])
        pltpu.sync_copy(den_sh.at[pl.ds(row0, NPT)],
                        den_hbm.at[rel, cid, pl.ds(row0, NPT)])


def _sc_gat(z0, z1, ss0, sd0, ss1, sd1, pk_r):
    mesh = plsc.VectorSubcoreMesh(core_axis_name="c", subcore_axis_name="s",
                                  num_cores=NC, num_subcores=NS)
    f = pl.kernel(
        _sc_body,
        out_type=[
            jax.ShapeDtypeStruct((2, NC, N_PAD, D), _f32),
            jax.ShapeDtypeStruct((2, NC, N_PAD), _f32),
        ],
        mesh=mesh,
        compiler_params=pltpu.CompilerParams(needs_layout_passes=False),
        scratch_types=[
            pltpu.VMEM((N,), _f32),
            pltpu.VMEM((N,), _f32),
            pltpu.VMEM((QCH, CHUNK), _i32),
            [pltpu.VMEM((CHUNK,), _i32) for _ in range(NBUF)],
            [pltpu.VMEM((CHUNK,), _i32) for _ in range(NBUF)],
            [pltpu.VMEM((CHUNK,), _f32) for _ in range(NBUF)],
            [pltpu.VMEM((CHUNK, D), _f32) for _ in range(NBUF)],
            [pltpu.SemaphoreType.DMA for _ in range(NBUF)],
            pltpu.VMEM_SHARED((N_PAD, D), _f32),
            pltpu.VMEM_SHARED((N_PAD,), _f32),
        ],
    )
    return f(z0, z1, ss0, sd0, ss1, sd1, pk_r)


def _tail_body(u_ref, den_ref, b_ref, o_ref):
    blk = o_ref.shape[0]
    den = jnp.sum(den_ref[...], axis=1)
    den = den.reshape(2, blk)
    den = jnp.where(den == 0.0, 1.0, den)
    u = u_ref[...]
    acc = (u[0, 0] + u[0, 1]) / den[0][:, None]
    acc = acc + (u[1, 0] + u[1, 1]) / den[1][:, None]
    o_ref[...] = acc + b_ref[...]


def _tc_tail(u_part, den_part, b):
    blk = 2048
    grid = N_PAD // blk
    return pl.pallas_call(
        _tail_body,
        grid=(grid,),
        in_specs=[
            pl.BlockSpec((2, NC, blk, D), lambda i: (0, 0, i, 0)),
            pl.BlockSpec((2, NC, blk // D, D), lambda i: (0, 0, i, 0)),
            pl.BlockSpec((1, D), lambda i: (0, 0)),
        ],
        out_specs=pl.BlockSpec((blk, D), lambda i: (i, 0)),
        out_shape=jax.ShapeDtypeStruct((N_PAD, D), _f32),
    )(u_part, den_part, b)


@jax.jit
def kernel(x, edge_index_rel0, edge_index_rel1, message_, W0, a_src0,
           a_dst0, W1, a_src1, a_dst1, b):
    A0 = jnp.stack([a_src0, a_dst0], axis=1)
    A1 = jnp.stack([a_src1, a_dst1], axis=1)
    z0, z1, s0p, s1p = _tc_front(x, W0, W1, A0, A1)
    ss0 = s0p[:, 0] + 0.0
    sd0 = s0p[:, 1] + 0.0
    ss1 = s1p[:, 0] + 0.0
    sd1 = s1p[:, 1] + 0.0

    pad = E_PAD - E
    ei = jnp.stack([edge_index_rel0, edge_index_rel1])
    ei = jnp.pad(ei, ((0, 0), (0, 0), (0, pad)))
    pk = (ei[:, 0] << 14) | ei[:, 1]
    pk_r = pk.reshape(2, NW, NQ, QCH, CHUNK)

    u_part, den_part = _sc_gat(z0, z1, ss0, sd0, ss1, sd1, pk_r)
    den_part = den_part.reshape(2, NC, DEN_R, D)
    return _tc_tail(u_part, den_part, b.reshape(1, D))[:N]


# speedup vs baseline: 21.8971x; 1.1765x over previous
import functools
import jax
import jax.numpy as jnp
from jax import lax
from jax.experimental import pallas as pl
from jax.experimental.pallas import tpu as pltpu
from jax.experimental.pallas import tpu_sc as plsc

N = 10000
D = 128
E = 160000

NC = 2
NS = 16
NW = NC * NS
EPT = 5120
E_PAD = NW * EPT
N_PAD = 10240
CHUNK = 32
NBUF = 4
NQ = 4
QCH = EPT // (NQ * CHUNK)
NP = QCH // NBUF
GRP = 16
GPC = CHUNK // GRP
DG = 128 // GRP
NPT = N_PAD // NS
DEN_R = N_PAD // D

_f32 = jnp.float32
_i32 = jnp.int32


def _front_body(x_ref, w0_ref, w1_ref, a0_ref, a1_ref,
                z0_ref, z1_ref, s0_ref, s1_ref):
    xb = x_ref[...]
    z0 = jnp.dot(xb, w0_ref[...], preferred_element_type=_f32)
    z1 = jnp.dot(xb, w1_ref[...], preferred_element_type=_f32)
    z0_ref[...] = z0
    z1_ref[...] = z1
    s0_ref[...] = jnp.dot(z0, a0_ref[...], preferred_element_type=_f32)
    s1_ref[...] = jnp.dot(z1, a1_ref[...], preferred_element_type=_f32)


def _tc_front(x, W0, W1, A0, A1):
    blk = 2000
    grid = N // blk
    return pl.pallas_call(
        _front_body,
        grid=(grid,),
        in_specs=[
            pl.BlockSpec((blk, D), lambda i: (i, 0)),
            pl.BlockSpec((D, D), lambda i: (0, 0)),
            pl.BlockSpec((D, D), lambda i: (0, 0)),
            pl.BlockSpec((D, 2), lambda i: (0, 0)),
            pl.BlockSpec((D, 2), lambda i: (0, 0)),
        ],
        out_specs=[
            pl.BlockSpec((blk, D), lambda i: (i, 0)),
            pl.BlockSpec((blk, D), lambda i: (i, 0)),
            pl.BlockSpec((blk, 2), lambda i: (i, 0)),
            pl.BlockSpec((blk, 2), lambda i: (i, 0)),
        ],
        out_shape=[
            jax.ShapeDtypeStruct((N, D), _f32),
            jax.ShapeDtypeStruct((N, D), _f32),
            jax.ShapeDtypeStruct((N, 2), _f32),
            jax.ShapeDtypeStruct((N, 2), _f32),
        ],
    )(x, W0, W1, A0, A1)


def _lrelu(v):
    return jnp.where(v >= 0.0, v, 0.2 * v)


def _sc_body(z0_hbm, z1_hbm, ss0_hbm, sd0_hbm, ss1_hbm, sd1_hbm, pk_hbm,
             u_hbm, den_hbm,
             s_src_v, s_dst_v, pk_v, srcb, dstb, exb, rowsb, rows, sems,
             u_sh, den_sh):
    cid = lax.axis_index("c")
    sid = lax.axis_index("s")
    wid = cid * NS + sid
    row0 = sid * NPT
    iota = lax.iota(_i32, GRP)
    zeros16 = jnp.zeros((GRP,), _f32)
    rows0 = rows[0]

    def _zrow(i, _):
        for j in range(DG):
            rows0[i, pl.ds(j * GRP, GRP)] = zeros16
        return 0

    for rel in range(2):
        z_hbm = (z0_hbm, z1_hbm)[rel]
        ss_hbm = (ss0_hbm, ss1_hbm)[rel]
        sd_hbm = (sd0_hbm, sd1_hbm)[rel]

        lax.fori_loop(0, CHUNK, _zrow, 0)
        ex0 = exb[0]
        for g in range(GPC):
            ex0[pl.ds(g * GRP, GRP)] = zeros16
        for k in range(NPT // CHUNK):
            pltpu.sync_copy(rows0, u_sh.at[pl.ds(row0 + k * CHUNK, CHUNK)])
            pltpu.sync_copy(ex0, den_sh.at[pl.ds(row0 + k * CHUNK, CHUNK)])

        pltpu.sync_copy(ss_hbm, s_src_v)
        pltpu.sync_copy(sd_hbm, s_dst_v)

        def _mx(i, c):
            a, bm = c
            return (jnp.maximum(a, s_src_v[pl.ds(i * GRP, GRP)]),
                    jnp.maximum(bm, s_dst_v[pl.ds(i * GRP, GRP)]))
        neg = jnp.full((GRP,), -3.0e38, _f32)
        mS, mD = lax.fori_loop(0, N // GRP, _mx, (neg, neg))

        def _allmax(v):
            for sh in (1, 2, 4, 8):
                ex0[pl.ds(0, GRP)] = v
                v = jnp.maximum(
                    v, plsc.load_gather(ex0, [jnp.bitwise_xor(iota, sh)]))
            return v
        mstar = _lrelu(_allmax(mS) + _allmax(mD))

        plsc.subcore_barrier()

        for q in range(NQ):
            pltpu.sync_copy(pk_hbm.at[rel, wid, q], pk_v)
            qbase = wid * EPT + q * QCH * CHUNK

            def _score(c, i):
                def _grp(g, _):
                    pv = pk_v[c, pl.ds(g * GRP, GRP)]
                    sv = lax.shift_right_logical(pv, 14)
                    dv = jnp.bitwise_and(pv, 16383)
                    srcb[i][pl.ds(g * GRP, GRP)] = sv
                    dstb[i][pl.ds(g * GRP, GRP)] = dv
                    a = plsc.load_gather(s_src_v, [sv])
                    bm = plsc.load_gather(s_dst_v, [dv])
                    e = _lrelu(a + bm)
                    ex = jnp.exp(e - mstar)
                    valid = (qbase + c * CHUNK + g * GRP + iota) < E
                    exb[i][pl.ds(g * GRP, GRP)] = jnp.where(valid, ex, 0.0)
                    return 0
                lax.fori_loop(0, GPC, _grp, 0)
                pltpu.sync_copy(exb[i], den_sh.at[dstb[i]], add=True)

            def _scale(i):
                def _s(r, _):
                    w = plsc.load_gather(exb[i], [jnp.full((GRP,), r, _i32)])
                    for g in range(DG // 2):
                        pw = rowsb[i][r, pl.ds(g * GRP, GRP)]
                        a = plsc.bitcast(lax.shift_left(pw, 16), _f32)
                        bh = jnp.bitwise_and(pw, jnp.int32(-65536))
                        b = plsc.bitcast(bh, _f32)
                        rows[i][r, pl.ds(32 * g, GRP)] = a * w
                        rows[i][r, pl.ds(32 * g + GRP, GRP)] = b * w
                    return 0
                lax.fori_loop(0, CHUNK, _s, 0)

            for i in range(NBUF):
                _score(i, i)
                pltpu.async_copy(z_hbm.at[srcb[i]], rowsb[i], sems[i])

            def _round(k, _):
                c0 = NBUF * k
                for i in range(NBUF):
                    pltpu.make_async_copy(z_hbm.at[srcb[i]], rowsb[i],
                                          sems[i]).wait()
                    _scale(i)
                    pltpu.async_copy(rows[i], u_sh.at[dstb[i]], sems[i],
                                     add=True)
                @pl.when(k < NP - 1)
                def _():
                    for i in range(NBUF):
                        pltpu.make_async_copy(rows[i], u_sh.at[dstb[i]],
                                              sems[i]).wait()
                        _score(c0 + NBUF + i, i)
                        pltpu.async_copy(z_hbm.at[srcb[i]], rowsb[i], sems[i])
                return 0

            lax.fori_loop(0, NP, _round, 0)

            for i in range(NBUF):
                pltpu.make_async_copy(rows[i], u_sh.at[dstb[i]],
                                      sems[i]).wait()

        plsc.subcore_barrier()

        pltpu.sync_copy(u_sh.at[pl.ds(row0, NPT)],
                        u_hbm.at[rel, cid, pl.ds(row0, NPT)])
        pltpu.sync_copy(den_sh.at[pl.ds(row0, NPT)],
                        den_hbm.at[rel, cid, pl.ds(row0, NPT)])


def _sc_gat(z0, z1, ss0, sd0, ss1, sd1, pk_r):
    mesh = plsc.VectorSubcoreMesh(core_axis_name="c", subcore_axis_name="s",
                                  num_cores=NC, num_subcores=NS)
    f = pl.kernel(
        _sc_body,
        out_type=[
            jax.ShapeDtypeStruct((2, NC, N_PAD, D), _f32),
            jax.ShapeDtypeStruct((2, NC, N_PAD), _f32),
        ],
        mesh=mesh,
        compiler_params=pltpu.CompilerParams(needs_layout_passes=False, use_tc_tiling_on_sc=False),
        scratch_types=[
            pltpu.VMEM((N,), _f32),
            pltpu.VMEM((N,), _f32),
            pltpu.VMEM((QCH, CHUNK), _i32),
            [pltpu.VMEM((CHUNK,), _i32) for _ in range(NBUF)],
            [pltpu.VMEM((CHUNK,), _i32) for _ in range(NBUF)],
            [pltpu.VMEM((CHUNK,), _f32) for _ in range(NBUF)],
            [pltpu.VMEM((CHUNK, D // 2), _i32) for _ in range(NBUF)],
            [pltpu.VMEM((CHUNK, D), _f32) for _ in range(NBUF)],
            [pltpu.SemaphoreType.DMA for _ in range(NBUF)],
            pltpu.VMEM_SHARED((N_PAD, D), _f32),
            pltpu.VMEM_SHARED((N_PAD,), _f32),
        ],
    )
    return f(z0, z1, ss0, sd0, ss1, sd1, pk_r)


def _tail_body(u_ref, den_ref, b_ref, o_ref):
    blk = o_ref.shape[0]
    den = jnp.sum(den_ref[...], axis=1)
    den = den.reshape(2, blk)
    den = jnp.where(den == 0.0, 1.0, den)
    u = u_ref[...]
    acc = (u[0, 0] + u[0, 1]) / den[0][:, None]
    acc = acc + (u[1, 0] + u[1, 1]) / den[1][:, None]
    o_ref[...] = acc + b_ref[...]


def _tc_tail(u_part, den_part, b):
    blk = 2048
    grid = N_PAD // blk
    return pl.pallas_call(
        _tail_body,
        grid=(grid,),
        in_specs=[
            pl.BlockSpec((2, NC, blk, D), lambda i: (0, 0, i, 0)),
            pl.BlockSpec((2, NC, blk // D, D), lambda i: (0, 0, i, 0)),
            pl.BlockSpec((1, D), lambda i: (0, 0)),
        ],
        out_specs=pl.BlockSpec((blk, D), lambda i: (i, 0)),
        out_shape=jax.ShapeDtypeStruct((N_PAD, D), _f32),
    )(u_part, den_part, b)


@jax.jit
def kernel(x, edge_index_rel0, edge_index_rel1, message_, W0, a_src0,
           a_dst0, W1, a_src1, a_dst1, b):
    A0 = jnp.stack([a_src0, a_dst0], axis=1)
    A1 = jnp.stack([a_src1, a_dst1], axis=1)
    z0, z1, s0p, s1p = _tc_front(x, W0, W1, A0, A1)

    def _pack_bf16(z):
        zb = z.astype(jnp.bfloat16)
        zp = zb.reshape(N, D // 32, 2, 16).transpose(0, 1, 3, 2)
        return lax.bitcast_convert_type(zp.reshape(N, D // 2, 2), _i32)
    z0p = _pack_bf16(z0)
    z1p = _pack_bf16(z1)
    ss0 = s0p[:, 0] + 0.0
    sd0 = s0p[:, 1] + 0.0
    ss1 = s1p[:, 0] + 0.0
    sd1 = s1p[:, 1] + 0.0

    pad = E_PAD - E
    ei = jnp.stack([edge_index_rel0, edge_index_rel1])
    ei = jnp.pad(ei, ((0, 0), (0, 0), (0, pad)))
    pk = (ei[:, 0] << 14) | ei[:, 1]
    pk_r = pk.reshape(2, NW, NQ, QCH, CHUNK)

    u_part, den_part = _sc_gat(z0p, z1p, ss0, sd0, ss1, sd1, pk_r)
    den_part = den_part.reshape(2, NC, DEN_R, D)
    return _tc_tail(u_part, den_part, b.reshape(1, D))[:N]
